# baseline (speedup 1.0000x reference)
import functools

import jax
import jax.numpy as jnp
from jax import lax
from jax.experimental import pallas as pl
from jax.experimental.pallas import tpu as pltpu

B = 2048
D = 256
K = 1024
ND = 8
H = D // 2

BM = 512
NBLK = B // BM


def _gelu_exact(x):
    return 0.5 * x * (1.0 + lax.erf(x * (2.0 ** -0.5)))


def _layernorm(x, g, b):
    mu = jnp.mean(x, axis=-1, keepdims=True)
    var = jnp.mean((x - mu) ** 2, axis=-1, keepdims=True)
    return (x - mu) / jnp.sqrt(var + 1e-5) * g + b


def _fused_body(x_ref, did_ref, w1_ref, b1_ref, g1_ref, be1_ref, w2_ref,
                b2_ref, pw1_ref, pb1_ref, pg_ref, pbe_ref, pw2_ref, pb2_ref,
                p_hbm, gp_hbm, enh_ref, w_hbm, p_vmem, g_vmem, m_vmem,
                w_vmem, psem, gsem, wsem):
    pltpu.make_async_copy(p_hbm, p_vmem, psem).start()
    pltpu.make_async_copy(gp_hbm, g_vmem, gsem).start()

    dot = functools.partial(jnp.dot, preferred_element_type=jnp.float32)

    def heads(x):
        h = dot(x, w1_ref[...]) + b1_ref[...]
        h = _layernorm(h, g1_ref[...], be1_ref[...])
        h = _gelu_exact(h)
        h = dot(h, w2_ref[...]) + b2_ref[...]
        nrm = jnp.sqrt(jnp.sum(h * h, axis=-1, keepdims=True))
        feats = h / jnp.maximum(nrm, 1e-12)
        t = dot(x, pw1_ref[...]) + pb1_ref[...]
        t = _layernorm(t, pg_ref[...], pbe_ref[...])
        t = _gelu_exact(t)
        logits = dot(t, pw2_ref[...]) + pb2_ref[...]
        m = jnp.max(logits, axis=-1, keepdims=True)
        e = jnp.exp(logits - m)
        return feats, e / jnp.sum(e, axis=-1, keepdims=True)

    per_block = []
    for blk in range(NBLK):
        rows = pl.ds(blk * BM, BM)
        feats, w = heads(x_ref[rows, :])
        w_vmem[rows, :] = w
        pltpu.make_async_copy(
            w_vmem.at[rows, :], w_hbm.at[rows, :], wsem).start()
        per_block.append((feats, w))

    pltpu.make_async_copy(p_hbm, p_vmem, psem).wait()
    pltpu.make_async_copy(gp_hbm, g_vmem, gsem).wait()
    g = g_vmem[...]
    for n in range(ND):
        sl = pl.ds(n * K, K)
        m_vmem[sl, :] = 0.12 * p_vmem[sl, :] + 0.08 * g

    for blk in range(NBLK):
        rows = pl.ds(blk * BM, BM)
        feats, w = per_block[blk]
        did = jnp.minimum(did_ref[blk, 0, :], ND - 1).reshape(BM, 1)
        acc = feats
        for n in range(ND):
            sel = (did == n).astype(jnp.float32)
            acc += sel * dot(w, m_vmem[pl.ds(n * K, K), :])
        enh_ref[rows, :] = acc

    for blk in range(NBLK):
        rows = pl.ds(blk * BM, BM)
        pltpu.make_async_copy(
            w_vmem.at[rows, :], w_hbm.at[rows, :], wsem).wait()


def kernel(features, domain_ids, ph_W1, ph_b1, ln1_g, ln1_b, ph_W2, ph_b2,
           pw_W1, pw_b1, pw_ln_g, pw_ln_b, pw_W2, pw_b2,
           domain_prototypes, global_prototypes):
    did3 = domain_ids.astype(jnp.int32).reshape(NBLK, 1, BM)

    def const(shape):
        return pl.BlockSpec(shape, lambda i: (0,) * len(shape))

    enhanced, w = pl.pallas_call(
        _fused_body,
        grid=(1,),
        in_specs=[
            const((B, D)),
            const((NBLK, 1, BM)),
            const((D, D)), const((1, D)), const((1, D)), const((1, D)),
            const((D, D)), const((1, D)),
            const((D, H)), const((1, H)), const((1, H)), const((1, H)),
            const((H, K)), const((1, K)),
            pl.BlockSpec(memory_space=pl.ANY),
            pl.BlockSpec(memory_space=pl.ANY),
        ],
        out_specs=[
            const((B, D)),
            pl.BlockSpec(memory_space=pl.ANY),
        ],
        out_shape=[
            jax.ShapeDtypeStruct((B, D), jnp.float32),
            jax.ShapeDtypeStruct((B, K), jnp.float32),
        ],
        scratch_shapes=[
            pltpu.VMEM((ND * K, D), jnp.float32),
            pltpu.VMEM((K, D), jnp.float32),
            pltpu.VMEM((ND * K, D), jnp.float32),
            pltpu.VMEM((B, K), jnp.float32),
            pltpu.SemaphoreType.DMA,
            pltpu.SemaphoreType.DMA,
            pltpu.SemaphoreType.DMA,
        ],
    )(features, did3, ph_W1, ph_b1.reshape(1, D), ln1_g.reshape(1, D),
      ln1_b.reshape(1, D), ph_W2, ph_b2.reshape(1, D),
      pw_W1, pw_b1.reshape(1, H), pw_ln_g.reshape(1, H),
      pw_ln_b.reshape(1, H), pw_W2, pw_b2.reshape(1, K),
      domain_prototypes.reshape(ND * K, D), global_prototypes)
    return (enhanced, w)


# speedup vs baseline: 1.0645x; 1.0645x over previous
import functools

import jax
import jax.numpy as jnp
from jax import lax
from jax.experimental import pallas as pl
from jax.experimental.pallas import tpu as pltpu

B = 2048
D = 256
K = 1024
ND = 8
H = D // 2

BM = 1024
NBLK = B // BM


def _gelu_exact(x):
    return 0.5 * x * (1.0 + lax.erf(x * (2.0 ** -0.5)))


def _layernorm(x, g, b):
    mu = jnp.mean(x, axis=-1, keepdims=True)
    var = jnp.mean((x - mu) ** 2, axis=-1, keepdims=True)
    return (x - mu) / jnp.sqrt(var + 1e-5) * g + b


def _fused_body(x_ref, did_ref, w1_ref, b1_ref, g1_ref, be1_ref, w2_ref,
                b2_ref, pw1_ref, pb1_ref, pg_ref, pbe_ref, pw2_ref, pb2_ref,
                p_hbm, gp_hbm, enh_ref, w_hbm, p_vmem, g_vmem, m_vmem,
                w_vmem, psem, gsem, wsem):
    pltpu.make_async_copy(p_hbm, p_vmem, psem).start()
    pltpu.make_async_copy(gp_hbm, g_vmem, gsem).start()

    dot = functools.partial(jnp.dot, preferred_element_type=jnp.float32)

    def heads(x):
        h = dot(x, w1_ref[...]) + b1_ref[...]
        h = _layernorm(h, g1_ref[...], be1_ref[...])
        h = _gelu_exact(h)
        h = dot(h, w2_ref[...]) + b2_ref[...]
        nrm = jnp.sqrt(jnp.sum(h * h, axis=-1, keepdims=True))
        feats = h / jnp.maximum(nrm, 1e-12)
        t = dot(x, pw1_ref[...]) + pb1_ref[...]
        t = _layernorm(t, pg_ref[...], pbe_ref[...])
        t = _gelu_exact(t)
        logits = dot(t, pw2_ref[...]) + pb2_ref[...]
        m = jnp.max(logits, axis=-1, keepdims=True)
        e = jnp.exp(logits - m)
        return feats, e / jnp.sum(e, axis=-1, keepdims=True)

    per_block = []
    for blk in range(NBLK):
        rows = pl.ds(blk * BM, BM)
        feats, w = heads(x_ref[rows, :])
        w_vmem[rows, :] = w
        pltpu.make_async_copy(
            w_vmem.at[rows, :], w_hbm.at[rows, :], wsem).start()
        per_block.append((feats, w))

    pltpu.make_async_copy(p_hbm, p_vmem, psem).wait()
    pltpu.make_async_copy(gp_hbm, g_vmem, gsem).wait()
    g = g_vmem[...]
    for n in range(ND):
        sl = pl.ds(n * K, K)
        m_vmem[sl, :] = 0.12 * p_vmem[sl, :] + 0.08 * g

    for blk in range(NBLK):
        rows = pl.ds(blk * BM, BM)
        feats, w = per_block[blk]
        did = jnp.minimum(did_ref[blk, 0, :], ND - 1).reshape(BM, 1)
        acc = feats
        for n in range(ND):
            sel = (did == n).astype(jnp.float32)
            acc += sel * dot(w, m_vmem[pl.ds(n * K, K), :])
        enh_ref[rows, :] = acc

    for blk in range(NBLK):
        rows = pl.ds(blk * BM, BM)
        pltpu.make_async_copy(
            w_vmem.at[rows, :], w_hbm.at[rows, :], wsem).wait()


def kernel(features, domain_ids, ph_W1, ph_b1, ln1_g, ln1_b, ph_W2, ph_b2,
           pw_W1, pw_b1, pw_ln_g, pw_ln_b, pw_W2, pw_b2,
           domain_prototypes, global_prototypes):
    did3 = domain_ids.astype(jnp.int32).reshape(NBLK, 1, BM)

    def const(shape):
        return pl.BlockSpec(shape, lambda i: (0,) * len(shape))

    enhanced, w = pl.pallas_call(
        _fused_body,
        grid=(1,),
        in_specs=[
            const((B, D)),
            const((NBLK, 1, BM)),
            const((D, D)), const((1, D)), const((1, D)), const((1, D)),
            const((D, D)), const((1, D)),
            const((D, H)), const((1, H)), const((1, H)), const((1, H)),
            const((H, K)), const((1, K)),
            pl.BlockSpec(memory_space=pl.ANY),
            pl.BlockSpec(memory_space=pl.ANY),
        ],
        out_specs=[
            const((B, D)),
            pl.BlockSpec(memory_space=pl.ANY),
        ],
        out_shape=[
            jax.ShapeDtypeStruct((B, D), jnp.float32),
            jax.ShapeDtypeStruct((B, K), jnp.float32),
        ],
        scratch_shapes=[
            pltpu.VMEM((ND * K, D), jnp.float32),
            pltpu.VMEM((K, D), jnp.float32),
            pltpu.VMEM((ND * K, D), jnp.float32),
            pltpu.VMEM((B, K), jnp.float32),
            pltpu.SemaphoreType.DMA,
            pltpu.SemaphoreType.DMA,
            pltpu.SemaphoreType.DMA,
        ],
    )(features, did3, ph_W1, ph_b1.reshape(1, D), ln1_g.reshape(1, D),
      ln1_b.reshape(1, D), ph_W2, ph_b2.reshape(1, D),
      pw_W1, pw_b1.reshape(1, H), pw_ln_g.reshape(1, H),
      pw_ln_b.reshape(1, H), pw_W2, pw_b2.reshape(1, K),
      domain_prototypes.reshape(ND * K, D), global_prototypes)
    return (enhanced, w)
